# baseline (speedup 1.0000x reference)
import functools

import jax
import jax.numpy as jnp
from jax import lax
from jax.experimental import pallas as pl
from jax.experimental.pallas import tpu as pltpu
from jax.experimental.pallas import tpu_sc as plsc

_NUM_FEATURES = 26
_FEATURE_SIZE = 100000
_BATCH = 4096
_EMBED_DIM = 32
_NC = 2
_NS = 16
_LANES = 16
_NW = _NC * _NS
_TOTAL = _BATCH * _NUM_FEATURES
_PER_W = _TOTAL // _NW
_NCH = 4
_CH = _PER_W // _NCH


def _sc_body(x_hbm, table_hbm, out_hbm, idx_v, rows_v, g_sems, s_sems):
    wid = lax.axis_index("s") * _NC + lax.axis_index("c")
    base = wid * _PER_W

    pltpu.sync_copy(x_hbm.at[pl.ds(base, _PER_W)], idx_v)

    lane = lax.iota(jnp.int32, _LANES)

    def offset_add(c):
        def body(i, _):
            j0 = c * _CH + i * _LANES
            feat = jnp.remainder(j0 + lane, _NUM_FEATURES)
            sl = pl.ds(j0, _LANES)
            idx_v[sl] = idx_v[sl] + feat * _FEATURE_SIZE
            return _
        lax.fori_loop(0, _CH // _LANES, body, None, unroll=8)

    offset_add(0)
    scatters = [None, None]
    for c in range(_NCH):
        b = c % 2
        if scatters[b] is not None:
            scatters[b].wait()
        g = pltpu.async_copy(
            table_hbm.at[idx_v.at[pl.ds(c * _CH, _CH)]],
            rows_v.at[b], g_sems.at[b])
        if c + 1 < _NCH:
            offset_add(c + 1)
        g.wait()
        scatters[b] = pltpu.async_copy(
            rows_v.at[b],
            out_hbm.at[pl.ds(base + c * _CH, _CH)], s_sems.at[b])
    for s in scatters:
        if s is not None:
            s.wait()


@jax.jit
def kernel(x, table):
    x_flat = x.reshape(_TOTAL).astype(jnp.int32)
    mesh = plsc.VectorSubcoreMesh(
        core_axis_name="c", subcore_axis_name="s",
        num_cores=_NC, num_subcores=_NS,
    )
    out = pl.kernel(
        _sc_body,
        out_type=jax.ShapeDtypeStruct((_TOTAL, _EMBED_DIM), jnp.float32),
        mesh=mesh,
        scratch_types=[
            pltpu.VMEM((_PER_W,), jnp.int32),
            pltpu.VMEM((2, _CH, _EMBED_DIM), jnp.float32),
            pltpu.SemaphoreType.DMA((2,)),
            pltpu.SemaphoreType.DMA((2,)),
        ],
        compiler_params=pltpu.CompilerParams(use_tc_tiling_on_sc=False),
    )(x_flat, table)
    return out.reshape(_BATCH, _NUM_FEATURES, _EMBED_DIM)


# speedup vs baseline: 1.5940x; 1.5940x over previous
import functools

import jax
import jax.numpy as jnp
from jax import lax
from jax.experimental import pallas as pl
from jax.experimental.pallas import tpu as pltpu
from jax.experimental.pallas import tpu_sc as plsc

_NUM_FEATURES = 26
_FEATURE_SIZE = 100000
_BATCH = 4096
_EMBED_DIM = 32
_NC = 2
_NS = 16
_NW = _NC * _NS
_BPW = _BATCH // _NW
_CB = 8
_NCH = _BPW // _CB


def _sc_body(x_hbm, table_hbm, out_hbm, x_v, rows_v, g_sems, s_sems):
    wid = lax.axis_index("s") * _NC + lax.axis_index("c")
    b0 = wid * _BPW

    pltpu.sync_copy(x_hbm.at[pl.ds(b0, _BPW)], x_v)

    scatters = [None, None]
    for c in range(_NCH):
        b = c % 2
        if scatters[b] is not None:
            scatters[b].wait()
        def issue(jb, _, b=b, c=c):
            row = c * _CB + jb
            w0 = x_v[row, pl.ds(0, 16)]
            w1 = x_v[row, pl.ds(_NUM_FEATURES - 16, 16)]
            for jf in range(_NUM_FEATURES):
                xv = w0[jf] if jf < 16 else w1[jf - (_NUM_FEATURES - 16)]
                r = xv + jf * _FEATURE_SIZE
                pltpu.async_copy(
                    table_hbm.at[r], rows_v.at[b, jb, jf], g_sems.at[b])
            return _

        lax.fori_loop(0, _CB, issue, None)

        def drain(j, _, b=b):
            pltpu.make_async_copy(
                table_hbm.at[0], rows_v.at[b, 0, 0], g_sems.at[b]).wait()
            return _

        lax.fori_loop(0, _CB * _NUM_FEATURES, drain, None, unroll=8)

        scatters[b] = pltpu.async_copy(
            rows_v.at[b], out_hbm.at[pl.ds(b0 + c * _CB, _CB)], s_sems.at[b])
    for s in scatters:
        if s is not None:
            s.wait()


@jax.jit
def kernel(x, table):
    mesh = plsc.VectorSubcoreMesh(
        core_axis_name="c", subcore_axis_name="s",
        num_cores=_NC, num_subcores=_NS,
    )
    return pl.kernel(
        _sc_body,
        out_type=jax.ShapeDtypeStruct(
            (_BATCH, _NUM_FEATURES, _EMBED_DIM), jnp.float32),
        mesh=mesh,
        scratch_types=[
            pltpu.VMEM((_BPW, _NUM_FEATURES), jnp.int32),
            pltpu.VMEM((2, _CB, _NUM_FEATURES, _EMBED_DIM), jnp.float32),
            pltpu.SemaphoreType.DMA((2,)),
            pltpu.SemaphoreType.DMA((2,)),
        ],
    )(x, table)


# speedup vs baseline: 1.6021x; 1.0051x over previous
import functools

import jax
import jax.numpy as jnp
from jax import lax
from jax.experimental import pallas as pl
from jax.experimental.pallas import tpu as pltpu
from jax.experimental.pallas import tpu_sc as plsc

_NUM_FEATURES = 26
_FEATURE_SIZE = 100000
_BATCH = 4096
_EMBED_DIM = 32
_NC = 2
_NS = 16
_NW = _NC * _NS
_BPW = _BATCH // _NW
_CB = 8
_NCH = _BPW // _CB
_NSEM = 4
_W16 = _NUM_FEATURES - 16


def _sc_body(x_hbm, table_hbm, out_hbm, x_v, rows_v, g_sems, s_sems):
    wid = lax.axis_index("s") * _NC + lax.axis_index("c")
    b0 = wid * _BPW

    pltpu.sync_copy(x_hbm.at[pl.ds(b0, _BPW)], x_v)

    def chunk(c, _):
        b = lax.rem(c, 2)

        @pl.when(c >= 2)
        def _wait_prev():
            pltpu.make_async_copy(
                rows_v.at[b], out_hbm.at[pl.ds(b0, _CB)], s_sems.at[b]
            ).wait()

        def issue(jb, _):
            row = c * _CB + jb
            w0 = x_v[row, pl.ds(0, 16)]
            w1 = x_v[row, pl.ds(_W16, 16)]
            for jf in range(_NUM_FEATURES):
                xv = w0[jf] if jf < 16 else w1[jf - _W16]
                r = xv + jf * _FEATURE_SIZE
                pltpu.async_copy(
                    table_hbm.at[r], rows_v.at[b, jb, jf],
                    g_sems.at[b, jf % _NSEM])
            return _

        lax.fori_loop(0, _CB, issue, None)

        for s in range(_NSEM):
            n_waits = _CB * len(
                [f for f in range(_NUM_FEATURES) if f % _NSEM == s])

            def drain(j, _, s=s):
                pltpu.make_async_copy(
                    table_hbm.at[0], rows_v.at[b, 0, 0],
                    g_sems.at[b, s]).wait()
                return _

            lax.fori_loop(0, n_waits, drain, None, unroll=8)

        pltpu.async_copy(
            rows_v.at[b], out_hbm.at[pl.ds(b0 + c * _CB, _CB)], s_sems.at[b])
        return _

    lax.fori_loop(0, _NCH, chunk, None)

    for b in range(2):
        pltpu.make_async_copy(
            rows_v.at[b], out_hbm.at[pl.ds(b0, _CB)], s_sems.at[b]).wait()


@jax.jit
def kernel(x, table):
    mesh = plsc.VectorSubcoreMesh(
        core_axis_name="c", subcore_axis_name="s",
        num_cores=_NC, num_subcores=_NS,
    )
    return pl.kernel(
        _sc_body,
        out_type=jax.ShapeDtypeStruct(
            (_BATCH, _NUM_FEATURES, _EMBED_DIM), jnp.float32),
        mesh=mesh,
        scratch_types=[
            pltpu.VMEM((_BPW, _NUM_FEATURES), jnp.int32),
            pltpu.VMEM((2, _CB, _NUM_FEATURES, _EMBED_DIM), jnp.float32),
            pltpu.SemaphoreType.DMA((2, _NSEM)),
            pltpu.SemaphoreType.DMA((2,)),
        ],
    )(x, table)
